# baseline (speedup 1.0000x reference)
import functools

import jax
import jax.numpy as jnp
from jax import lax
from jax.experimental import pallas as pl
from jax.experimental.pallas import tpu as pltpu
from jax.experimental.pallas import tpu_sc as plsc

N = 10000
NP = 10240
D = 128
E = 320000
NC = 2
NS = 16
NW = NC * NS
CPS = D // NC
CH = 128
TCH = 160
EPT = TCH * CH
EP = NS * EPT
DCH = 80
EPW = E // NW
NCH = EPW // DCH
RPT = NP // NS
ZR = 128

_mesh = plsc.VectorSubcoreMesh(core_axis_name="c", subcore_axis_name="s")
_sc_params = pltpu.CompilerParams(use_tc_tiling_on_sc=False)


@functools.partial(
    pl.kernel,
    out_type=jax.ShapeDtypeStruct((NC, 1, NP), jnp.float32),
    mesh=_mesh,
    scratch_types=[
        pltpu.VMEM((DCH,), jnp.float32),
        pltpu.VMEM((NCH, DCH), jnp.int32),
        pltpu.VMEM((RPT,), jnp.float32),
        pltpu.VMEM_SHARED((NP,), jnp.float32),
    ],
    compiler_params=_sc_params,
)
def _deg_kernel(dst_hbm, degp_hbm, ones_v, idx_v, zv, deg_sh):
    cid = lax.axis_index("c")
    sid = lax.axis_index("s")
    wid = sid * NC + cid

    for i in range(DCH // 16):
        ones_v[pl.ds(i * 16, 16)] = jnp.ones((16,), jnp.float32)

    def _zz(i, _):
        zv[pl.ds(i * 16, 16)] = jnp.zeros((16,), jnp.float32)
        return 0

    lax.fori_loop(0, RPT // 16, _zz, 0)
    pltpu.sync_copy(zv, deg_sh.at[pl.ds(sid * RPT, RPT)])
    plsc.subcore_barrier()

    pltpu.sync_copy(dst_hbm.at[wid], idx_v)

    def _body(j, _):
        pltpu.sync_copy(ones_v, deg_sh.at[idx_v.at[j]], add=True)
        return 0

    lax.fori_loop(0, NCH, _body, 0)
    plsc.subcore_barrier()

    pltpu.sync_copy(deg_sh.at[pl.ds(sid * RPT, RPT)],
                    degp_hbm.at[cid, 0, pl.ds(sid * RPT, RPT)])


@functools.partial(
    pl.kernel,
    out_type=jax.ShapeDtypeStruct((NC, NP, CPS), jnp.float32),
    mesh=_mesh,
    scratch_types=[
        pltpu.VMEM((TCH, CH), jnp.int32),
        pltpu.VMEM((TCH, CH), jnp.int32),
        pltpu.VMEM((2, CH, CPS), jnp.float32),
        pltpu.VMEM((ZR, CPS), jnp.float32),
        pltpu.VMEM_SHARED((NP, CPS), jnp.float32),
        pltpu.SemaphoreType.DMA((2,)),
    ],
    compiler_params=_sc_params,
)
def _agg_kernel(ghf_hbm, srcb_hbm, dst_hbm, agg_hbm,
                sidx_v, didx_v, rows_v, zv, acc_sh, gsem):
    cid = lax.axis_index("c")
    sid = lax.axis_index("s")

    def _zz(i, _):
        for c in range(CPS // 16):
            zv[i, pl.ds(c * 16, 16)] = jnp.zeros((16,), jnp.float32)
        return 0

    lax.fori_loop(0, ZR, _zz, 0)
    for t in range(RPT // ZR):
        pltpu.sync_copy(zv, acc_sh.at[pl.ds(sid * RPT + t * ZR, ZR)])
    plsc.subcore_barrier()

    pltpu.sync_copy(srcb_hbm.at[cid, sid], sidx_v)
    pltpu.sync_copy(dst_hbm.at[sid], didx_v)

    for b in range(2):
        pltpu.async_copy(ghf_hbm.at[sidx_v.at[b]], rows_v.at[b], gsem.at[b])

    def _step(j, b, issue_gather):
        pltpu.make_async_copy(
            ghf_hbm.at[sidx_v.at[j]], rows_v.at[b], gsem.at[b]).wait()
        pltpu.sync_copy(rows_v.at[b], acc_sh.at[didx_v.at[j]], add=True)
        if issue_gather:
            pltpu.async_copy(
                ghf_hbm.at[sidx_v.at[j + 2]], rows_v.at[b], gsem.at[b])

    def _pair(p, _):
        for b in range(2):
            _step(2 * p + b, b, True)
        return 0

    lax.fori_loop(0, TCH // 2 - 1, _pair, 0)
    for b in range(2):
        _step(TCH - 2 + b, b, False)
    plsc.subcore_barrier()

    pltpu.sync_copy(acc_sh.at[pl.ds(sid * RPT, RPT)],
                    agg_hbm.at[cid, pl.ds(sid * RPT, RPT)])


BLK = 2000
GRID = N // BLK


def _dinv(degt_ref):
    deg = degt_ref[:, 0:1] + degt_ref[:, 1:2] + 1.0
    return lax.rsqrt(deg)


def _tc1_body(degt_ref, x_ref, w_ref, g_ref):
    h = jnp.dot(x_ref[...], w_ref[0], preferred_element_type=jnp.float32)
    g_ref[0] = h * _dinv(degt_ref)


def _tc2_body(degt_ref, agg_ref, g_ref, b_ref, w_ref, out_ref):
    dinv = _dinv(degt_ref)
    a = jnp.concatenate([agg_ref[0] + g_ref[0], agg_ref[1] + g_ref[1]], axis=1)
    h1 = jnp.maximum(a * dinv + b_ref[...], 0.0)
    h2 = jnp.dot(h1, w_ref[0], preferred_element_type=jnp.float32)
    out_ref[0] = h2 * dinv


def _tc3_body(degt_ref, agg_ref, g_ref, b_ref, out_ref):
    dinv = _dinv(degt_ref)
    a = jnp.concatenate([agg_ref[0] + g_ref[0], agg_ref[1] + g_ref[1]], axis=1)
    out_ref[...] = a * dinv + b_ref[...]


_deg2_spec = pl.BlockSpec((BLK, NC), lambda j, i: (i, 0))
_deg1_spec = pl.BlockSpec((BLK, NC), lambda i: (i, 0))
_x_spec = pl.BlockSpec((BLK, D), lambda j, i: (i, 0))
_whalf_spec = pl.BlockSpec((1, D, CPS), lambda j, i: (j, 0, 0))
_ghalf_out_spec = pl.BlockSpec((1, BLK, CPS), lambda j, i: (j, i, 0))
_pair2_spec = pl.BlockSpec((NC, BLK, CPS), lambda j, i: (0, i, 0))
_pair1_spec = pl.BlockSpec((NC, BLK, CPS), lambda i: (0, i, 0))
_bias2_spec = pl.BlockSpec((1, D), lambda j, i: (0, 0))
_bias1_spec = pl.BlockSpec((1, D), lambda i: (0, 0))
_out_spec = pl.BlockSpec((BLK, D), lambda i: (i, 0))

_tc1 = pl.pallas_call(
    _tc1_body,
    grid=(NC, GRID),
    in_specs=[_deg2_spec, _x_spec, _whalf_spec],
    out_specs=_ghalf_out_spec,
    out_shape=jax.ShapeDtypeStruct((NC, N, CPS), jnp.float32),
)

_tc2 = pl.pallas_call(
    _tc2_body,
    grid=(NC, GRID),
    in_specs=[_deg2_spec, _pair2_spec, _pair2_spec, _bias2_spec, _whalf_spec],
    out_specs=_ghalf_out_spec,
    out_shape=jax.ShapeDtypeStruct((NC, N, CPS), jnp.float32),
)

_tc3 = pl.pallas_call(
    _tc3_body,
    grid=(GRID,),
    in_specs=[_deg1_spec, _pair1_spec, _pair1_spec, _bias1_spec],
    out_specs=_out_spec,
    out_shape=jax.ShapeDtypeStruct((N, D), jnp.float32),
)


def kernel(x, edge_index, W1, b1, W2, b2):
    ei = edge_index.astype(jnp.int32)
    pad = EP - E
    srcp = jnp.concatenate([ei[0], jnp.zeros((pad,), jnp.int32)])
    dstp = jnp.concatenate([ei[1], jnp.full((pad,), N, jnp.int32)])
    src3 = srcp.reshape(NS, TCH, CH)
    srcb = jnp.stack([src3, src3 + N])
    dst3 = dstp.reshape(NS, TCH, CH)
    dstw = ei[1].reshape(NW, NCH, DCH)
    b1r = b1.reshape(1, D)
    b2r = b2.reshape(1, D)
    W1s = jnp.stack([W1[:, :CPS], W1[:, CPS:]])
    W2s = jnp.stack([W2[:, :CPS], W2[:, CPS:]])

    degp = _deg_kernel(dstw)
    degt = degp.reshape(NC, NP)[:, :N].T

    g1 = _tc1(degt, x, W1s)
    agg1 = _agg_kernel(g1.reshape(NC * N, CPS), srcb, dst3)
    g2 = _tc2(degt, agg1, g1, b1r, W2s)
    agg2 = _agg_kernel(g2.reshape(NC * N, CPS), srcb, dst3)
    out = _tc3(degt, agg2, g2, b2r)
    return out


# speedup vs baseline: 1.0062x; 1.0062x over previous
import functools

import jax
import jax.numpy as jnp
from jax import lax
from jax.experimental import pallas as pl
from jax.experimental.pallas import tpu as pltpu
from jax.experimental.pallas import tpu_sc as plsc

N = 10000
NP = 10240
D = 128
E = 320000
NC = 2
NS = 16
NW = NC * NS
CPS = D // NC
CH = 128
TCH = 160
EPT = TCH * CH
EP = NS * EPT
DCH = 80
EPW = E // NW
NCH = EPW // DCH
RPT = NP // NS
ZR = 128

_mesh = plsc.VectorSubcoreMesh(core_axis_name="c", subcore_axis_name="s")
_sc_params = pltpu.CompilerParams(use_tc_tiling_on_sc=False)


@functools.partial(
    pl.kernel,
    out_type=jax.ShapeDtypeStruct((NC, 1, NP), jnp.float32),
    mesh=_mesh,
    scratch_types=[
        pltpu.VMEM((DCH,), jnp.float32),
        pltpu.VMEM((NCH, DCH), jnp.int32),
        pltpu.VMEM((RPT,), jnp.float32),
        pltpu.VMEM_SHARED((NP,), jnp.float32),
    ],
    compiler_params=_sc_params,
)
def _deg_kernel(dst_hbm, degp_hbm, ones_v, idx_v, zv, deg_sh):
    cid = lax.axis_index("c")
    sid = lax.axis_index("s")
    wid = sid * NC + cid

    for i in range(DCH // 16):
        ones_v[pl.ds(i * 16, 16)] = jnp.ones((16,), jnp.float32)

    def _zz(i, _):
        zv[pl.ds(i * 16, 16)] = jnp.zeros((16,), jnp.float32)
        return 0

    lax.fori_loop(0, RPT // 16, _zz, 0)
    pltpu.sync_copy(zv, deg_sh.at[pl.ds(sid * RPT, RPT)])
    plsc.subcore_barrier()

    pltpu.sync_copy(dst_hbm.at[wid], idx_v)

    def _body(j, _):
        pltpu.sync_copy(ones_v, deg_sh.at[idx_v.at[j]], add=True)
        return 0

    lax.fori_loop(0, NCH, _body, 0)
    plsc.subcore_barrier()

    pltpu.sync_copy(deg_sh.at[pl.ds(sid * RPT, RPT)],
                    degp_hbm.at[cid, 0, pl.ds(sid * RPT, RPT)])


@functools.partial(
    pl.kernel,
    out_type=jax.ShapeDtypeStruct((NC, NP, CPS), jnp.float32),
    mesh=_mesh,
    scratch_types=[
        pltpu.VMEM((TCH, CH), jnp.int32),
        pltpu.VMEM((TCH, CH), jnp.int32),
        pltpu.VMEM((2, CH, CPS), jnp.float32),
        pltpu.VMEM((ZR, CPS), jnp.float32),
        pltpu.VMEM_SHARED((NP, CPS), jnp.float32),
        pltpu.SemaphoreType.DMA((2,)),
    ],
    compiler_params=_sc_params,
)
def _agg_kernel(ghf_hbm, srcb_hbm, dst_hbm, agg_hbm,
                sidx_v, didx_v, rows_v, zv, acc_sh, gsem):
    cid = lax.axis_index("c")
    sid = lax.axis_index("s")

    def _zz(i, _):
        for c in range(CPS // 16):
            zv[i, pl.ds(c * 16, 16)] = jnp.zeros((16,), jnp.float32)
        return 0

    lax.fori_loop(0, ZR, _zz, 0)
    for t in range(RPT // ZR):
        pltpu.sync_copy(zv, acc_sh.at[pl.ds(sid * RPT + t * ZR, ZR)])
    plsc.subcore_barrier()

    pltpu.sync_copy(srcb_hbm.at[cid, sid], sidx_v)
    pltpu.sync_copy(dst_hbm.at[sid], didx_v)

    for b in range(2):
        pltpu.async_copy(ghf_hbm.at[sidx_v.at[b]], rows_v.at[b], gsem.at[b])

    def _step(j, b, issue_gather):
        pltpu.make_async_copy(
            ghf_hbm.at[sidx_v.at[j]], rows_v.at[b], gsem.at[b]).wait()
        pltpu.sync_copy(rows_v.at[b], acc_sh.at[didx_v.at[j]], add=True)
        if issue_gather:
            pltpu.async_copy(
                ghf_hbm.at[sidx_v.at[j + 2]], rows_v.at[b], gsem.at[b])

    def _pair(p, _):
        for b in range(2):
            _step(2 * p + b, b, True)
        return 0

    lax.fori_loop(0, TCH // 2 - 1, _pair, 0)
    for b in range(2):
        _step(TCH - 2 + b, b, False)
    plsc.subcore_barrier()

    pltpu.sync_copy(acc_sh.at[pl.ds(sid * RPT, RPT)],
                    agg_hbm.at[cid, pl.ds(sid * RPT, RPT)])


BLK = 2000
GRID = N // BLK


def _dinv(degt_ref):
    deg = degt_ref[:, 0:1] + degt_ref[:, 1:2] + 1.0
    return lax.rsqrt(deg)


def _tc1_body(degt_ref, x_ref, w_ref, g_ref):
    h = jnp.dot(x_ref[...], w_ref[0], preferred_element_type=jnp.float32)
    g_ref[0] = h * _dinv(degt_ref)


def _tc2_body(degt_ref, agg_ref, g_ref, b_ref, w_ref, out_ref):
    dinv = _dinv(degt_ref)
    a = jnp.concatenate([agg_ref[0] + g_ref[0], agg_ref[1] + g_ref[1]], axis=1)
    h1 = jnp.maximum(a * dinv + b_ref[...], 0.0)
    h2 = jnp.dot(h1, w_ref[0], preferred_element_type=jnp.float32)
    out_ref[0] = h2 * dinv


def _tc3_body(degt_ref, agg_ref, g_ref, b_ref, out_ref):
    dinv = _dinv(degt_ref)
    a = jnp.concatenate([agg_ref[0] + g_ref[0], agg_ref[1] + g_ref[1]], axis=1)
    out_ref[...] = a * dinv + b_ref[...]


_deg2_spec = pl.BlockSpec((BLK, NC), lambda j, i: (i, 0))
_deg1_spec = pl.BlockSpec((BLK, NC), lambda i: (i, 0))
_x_spec = pl.BlockSpec((BLK, D), lambda j, i: (i, 0))
_whalf_spec = pl.BlockSpec((1, D, CPS), lambda j, i: (j, 0, 0))
_ghalf_out_spec = pl.BlockSpec((1, BLK, CPS), lambda j, i: (j, i, 0))
_pair2_spec = pl.BlockSpec((NC, BLK, CPS), lambda j, i: (0, i, 0))
_pair1_spec = pl.BlockSpec((NC, BLK, CPS), lambda i: (0, i, 0))
_bias2_spec = pl.BlockSpec((1, D), lambda j, i: (0, 0))
_bias1_spec = pl.BlockSpec((1, D), lambda i: (0, 0))
_out_spec = pl.BlockSpec((BLK, D), lambda i: (i, 0))

_tc1 = pl.pallas_call(
    _tc1_body,
    grid=(NC, GRID),
    in_specs=[_deg2_spec, _x_spec, _whalf_spec],
    out_specs=_ghalf_out_spec,
    out_shape=jax.ShapeDtypeStruct((NC, N, CPS), jnp.float32),
)

_tc2 = pl.pallas_call(
    _tc2_body,
    grid=(NC, GRID),
    in_specs=[_deg2_spec, _pair2_spec, _pair2_spec, _bias2_spec, _whalf_spec],
    out_specs=_ghalf_out_spec,
    out_shape=jax.ShapeDtypeStruct((NC, N, CPS), jnp.float32),
)

_tc3 = pl.pallas_call(
    _tc3_body,
    grid=(GRID,),
    in_specs=[_deg1_spec, _pair1_spec, _pair1_spec, _bias1_spec],
    out_specs=_out_spec,
    out_shape=jax.ShapeDtypeStruct((N, D), jnp.float32),
)


def kernel(x, edge_index, W1, b1, W2, b2):
    ei = edge_index.astype(jnp.int32)
    pad = EP - E
    srcp = jnp.concatenate([ei[0], jnp.zeros((pad,), jnp.int32)])
    dstp = jnp.concatenate(
        [ei[1], N + jnp.arange(pad, dtype=jnp.int32) % (NP - N)])
    src3 = srcp.reshape(NS, TCH, CH)
    srcb = jnp.stack([src3, src3 + N])
    dst3 = dstp.reshape(NS, TCH, CH)
    dstw = ei[1].reshape(NW, NCH, DCH)
    b1r = b1.reshape(1, D)
    b2r = b2.reshape(1, D)
    W1s = jnp.stack([W1[:, :CPS], W1[:, CPS:]])
    W2s = jnp.stack([W2[:, :CPS], W2[:, CPS:]])

    degp = _deg_kernel(dstw)
    degt = degp.reshape(NC, NP)[:, :N].T

    g1 = _tc1(degt, x, W1s)
    agg1 = _agg_kernel(g1.reshape(NC * N, CPS), srcb, dst3)
    g2 = _tc2(degt, agg1, g1, b1r, W2s)
    agg2 = _agg_kernel(g2.reshape(NC * N, CPS), srcb, dst3)
    out = _tc3(degt, agg2, g2, b2r)
    return out


# speedup vs baseline: 1.3809x; 1.3724x over previous
import functools

import jax
import jax.numpy as jnp
from jax import lax
from jax.experimental import pallas as pl
from jax.experimental.pallas import tpu as pltpu
from jax.experimental.pallas import tpu_sc as plsc

N = 10000
NP = 10240
D = 128
E = 320000
NC = 2
NS = 16
NW = NC * NS
CPS = D // NC
CH = 128
TCH = 158
EPT = TCH * CH
EP = NS * EPT
DCH = 80
EPW = E // NW
NCH = EPW // DCH
RPT = NP // NS
ZR = 128

_mesh = plsc.VectorSubcoreMesh(core_axis_name="c", subcore_axis_name="s")
_sc_params = pltpu.CompilerParams(use_tc_tiling_on_sc=False)


@functools.partial(
    pl.kernel,
    out_type=jax.ShapeDtypeStruct((NC, 1, NP), jnp.float32),
    mesh=_mesh,
    scratch_types=[
        pltpu.VMEM((DCH,), jnp.float32),
        pltpu.VMEM((NCH, DCH), jnp.int32),
        pltpu.VMEM((RPT,), jnp.float32),
        pltpu.VMEM_SHARED((NP,), jnp.float32),
    ],
    compiler_params=_sc_params,
)
def _deg_kernel(dst_hbm, degp_hbm, ones_v, idx_v, zv, deg_sh):
    cid = lax.axis_index("c")
    sid = lax.axis_index("s")
    wid = sid * NC + cid

    for i in range(DCH // 16):
        ones_v[pl.ds(i * 16, 16)] = jnp.ones((16,), jnp.float32)

    def _zz(i, _):
        zv[pl.ds(i * 16, 16)] = jnp.zeros((16,), jnp.float32)
        return 0

    lax.fori_loop(0, RPT // 16, _zz, 0)
    pltpu.sync_copy(zv, deg_sh.at[pl.ds(sid * RPT, RPT)])
    plsc.subcore_barrier()

    pltpu.sync_copy(dst_hbm.at[wid], idx_v)

    def _body(j, _):
        pltpu.sync_copy(ones_v, deg_sh.at[idx_v.at[j]], add=True)
        return 0

    lax.fori_loop(0, NCH, _body, 0)
    plsc.subcore_barrier()

    pltpu.sync_copy(deg_sh.at[pl.ds(sid * RPT, RPT)],
                    degp_hbm.at[cid, 0, pl.ds(sid * RPT, RPT)])


@functools.partial(
    pl.kernel,
    out_type=jax.ShapeDtypeStruct((NC, NP, CPS), jnp.float32),
    mesh=_mesh,
    scratch_types=[
        pltpu.VMEM((TCH, CH), jnp.int32),
        pltpu.VMEM((TCH, CH), jnp.int32),
        pltpu.VMEM((2, CH, CPS), jnp.float32),
        pltpu.VMEM((ZR, CPS), jnp.float32),
        pltpu.VMEM_SHARED((NP, CPS), jnp.float32),
        pltpu.SemaphoreType.DMA((2,)),
    ],
    compiler_params=_sc_params,
)
def _agg_kernel(ghf_hbm, srcb_hbm, dst_hbm, agg_hbm,
                sidx_v, didx_v, rows_v, zv, acc_sh, gsem):
    cid = lax.axis_index("c")
    sid = lax.axis_index("s")

    def _zz(i, _):
        for c in range(CPS // 16):
            zv[i, pl.ds(c * 16, 16)] = jnp.zeros((16,), jnp.float32)
        return 0

    lax.fori_loop(0, ZR, _zz, 0)
    for t in range(RPT // ZR):
        pltpu.sync_copy(zv, acc_sh.at[pl.ds(sid * RPT + t * ZR, ZR)])
    plsc.subcore_barrier()

    pltpu.sync_copy(srcb_hbm.at[cid, sid], sidx_v)
    pltpu.sync_copy(dst_hbm.at[sid], didx_v)

    for b in range(2):
        pltpu.async_copy(ghf_hbm.at[sidx_v.at[b]], rows_v.at[b], gsem.at[b])

    def _step(j, b, issue_gather):
        pltpu.make_async_copy(
            ghf_hbm.at[sidx_v.at[j]], rows_v.at[b], gsem.at[b]).wait()
        pltpu.sync_copy(rows_v.at[b], acc_sh.at[didx_v.at[j]], add=True)
        if issue_gather:
            pltpu.async_copy(
                ghf_hbm.at[sidx_v.at[j + 2]], rows_v.at[b], gsem.at[b])

    def _pair(p, _):
        for b in range(2):
            _step(2 * p + b, b, True)
        return 0

    lax.fori_loop(0, TCH // 2 - 1, _pair, 0)
    for b in range(2):
        _step(TCH - 2 + b, b, False)
    plsc.subcore_barrier()

    pltpu.sync_copy(acc_sh.at[pl.ds(sid * RPT, RPT)],
                    agg_hbm.at[cid, pl.ds(sid * RPT, RPT)])


BLK = 2000
GRID = N // BLK


def _dinv(degt_ref):
    deg = degt_ref[:, 0:1] + degt_ref[:, 1:2] + 1.0
    return lax.rsqrt(deg)


def _tc1_body(degt_ref, x_ref, w_ref, g_ref):
    h = jnp.dot(x_ref[...], w_ref[0], preferred_element_type=jnp.float32)
    g_ref[0] = h * _dinv(degt_ref)


def _tc2_body(degt_ref, agg_ref, g_ref, b_ref, w_ref, out_ref):
    dinv = _dinv(degt_ref)
    a = jnp.concatenate([agg_ref[0] + g_ref[0], agg_ref[1] + g_ref[1]], axis=1)
    h1 = jnp.maximum(a * dinv + b_ref[...], 0.0)
    h2 = jnp.dot(h1, w_ref[0], preferred_element_type=jnp.float32)
    out_ref[0] = h2 * dinv


def _tc3_body(degt_ref, agg_ref, g_ref, b_ref, out_ref):
    dinv = _dinv(degt_ref)
    a = jnp.concatenate([agg_ref[0] + g_ref[0], agg_ref[1] + g_ref[1]], axis=1)
    out_ref[...] = a * dinv + b_ref[...]


_deg2_spec = pl.BlockSpec((BLK, NC), lambda j, i: (i, 0))
_deg1_spec = pl.BlockSpec((BLK, NC), lambda i: (i, 0))
_x_spec = pl.BlockSpec((BLK, D), lambda j, i: (i, 0))
_whalf_spec = pl.BlockSpec((1, D, CPS), lambda j, i: (j, 0, 0))
_ghalf_out_spec = pl.BlockSpec((1, BLK, CPS), lambda j, i: (j, i, 0))
_pair2_spec = pl.BlockSpec((NC, BLK, CPS), lambda j, i: (0, i, 0))
_pair1_spec = pl.BlockSpec((NC, BLK, CPS), lambda i: (0, i, 0))
_bias2_spec = pl.BlockSpec((1, D), lambda j, i: (0, 0))
_bias1_spec = pl.BlockSpec((1, D), lambda i: (0, 0))
_out_spec = pl.BlockSpec((BLK, D), lambda i: (i, 0))

_tc1 = pl.pallas_call(
    _tc1_body,
    grid=(NC, GRID),
    in_specs=[_deg2_spec, _x_spec, _whalf_spec],
    out_specs=_ghalf_out_spec,
    out_shape=jax.ShapeDtypeStruct((NC, N, CPS), jnp.float32),
)

_tc2 = pl.pallas_call(
    _tc2_body,
    grid=(NC, GRID),
    in_specs=[_deg2_spec, _pair2_spec, _pair2_spec, _bias2_spec, _whalf_spec],
    out_specs=_ghalf_out_spec,
    out_shape=jax.ShapeDtypeStruct((NC, N, CPS), jnp.float32),
)

_tc3 = pl.pallas_call(
    _tc3_body,
    grid=(GRID,),
    in_specs=[_deg1_spec, _pair1_spec, _pair1_spec, _bias1_spec],
    out_specs=_out_spec,
    out_shape=jax.ShapeDtypeStruct((N, D), jnp.float32),
)


def kernel(x, edge_index, W1, b1, W2, b2):
    ei = edge_index.astype(jnp.int32)
    pad = EP - E
    srcp = jnp.concatenate([ei[0], jnp.zeros((pad,), jnp.int32)])
    dstp = jnp.concatenate(
        [ei[1], N + jnp.arange(pad, dtype=jnp.int32) % (NP - N)])
    src3 = srcp.reshape(NS, TCH, CH)
    srcb = jnp.stack([src3, src3 + N])
    dst3 = dstp.reshape(NS, TCH, CH)
    dstw = ei[1].reshape(NW, NCH, DCH)
    b1r = b1.reshape(1, D)
    b2r = b2.reshape(1, D)
    W1s = jnp.stack([W1[:, :CPS], W1[:, CPS:]])
    W2s = jnp.stack([W2[:, :CPS], W2[:, CPS:]])

    degp = _deg_kernel(dstw)
    degt = degp.reshape(NC, NP)[:, :N].T

    g1 = _tc1(degt, x, W1s)
    agg1 = _agg_kernel(g1.reshape(NC * N, CPS), srcb, dst3)
    g2 = _tc2(degt, agg1, g1, b1r, W2s)
    agg2 = _agg_kernel(g2.reshape(NC * N, CPS), srcb, dst3)
    out = _tc3(degt, agg2, g2, b2r)
    return out


# speedup vs baseline: 1.4761x; 1.0689x over previous
import functools

import jax
import jax.numpy as jnp
from jax import lax
from jax.experimental import pallas as pl
from jax.experimental.pallas import tpu as pltpu
from jax.experimental.pallas import tpu_sc as plsc

N = 10000
NP = 10240
D = 128
E = 320000
NC = 2
NS = 16
NW = NC * NS
CPS = D // NC
CH = 128
TCH = 158
EPT = TCH * CH
EP = NS * EPT
DCH = 80
EPW = E // NW
NCH = EPW // DCH
RPT = NP // NS
ZR = 128

_mesh = plsc.VectorSubcoreMesh(core_axis_name="c", subcore_axis_name="s")
_sc_params = pltpu.CompilerParams(use_tc_tiling_on_sc=False)


@functools.partial(
    pl.kernel,
    out_type=jax.ShapeDtypeStruct((NC, 1, NP), jnp.float32),
    mesh=_mesh,
    scratch_types=[
        pltpu.VMEM((DCH,), jnp.float32),
        pltpu.VMEM((NCH, DCH), jnp.int32),
        pltpu.VMEM((RPT,), jnp.float32),
        pltpu.VMEM_SHARED((NP,), jnp.float32),
    ],
    compiler_params=_sc_params,
)
def _deg_kernel(dst_hbm, degp_hbm, ones_v, idx_v, zv, deg_sh):
    cid = lax.axis_index("c")
    sid = lax.axis_index("s")
    wid = sid * NC + cid

    for i in range(DCH // 16):
        ones_v[pl.ds(i * 16, 16)] = jnp.ones((16,), jnp.float32)

    def _zz(i, _):
        zv[pl.ds(i * 16, 16)] = jnp.zeros((16,), jnp.float32)
        return 0

    lax.fori_loop(0, RPT // 16, _zz, 0)
    pltpu.sync_copy(zv, deg_sh.at[pl.ds(sid * RPT, RPT)])
    plsc.subcore_barrier()

    pltpu.sync_copy(dst_hbm.at[wid], idx_v)

    def _body(j, _):
        pltpu.sync_copy(ones_v, deg_sh.at[idx_v.at[j]], add=True)
        return 0

    lax.fori_loop(0, NCH, _body, 0)
    plsc.subcore_barrier()

    pltpu.sync_copy(deg_sh.at[pl.ds(sid * RPT, RPT)],
                    degp_hbm.at[cid, 0, pl.ds(sid * RPT, RPT)])


@functools.partial(
    pl.kernel,
    out_type=jax.ShapeDtypeStruct((NC, NP, CPS), jnp.float32),
    mesh=_mesh,
    scratch_types=[
        pltpu.VMEM((TCH // 2, 2 * CH), jnp.int32),
        pltpu.VMEM((TCH // 2, 2 * CH), jnp.int32),
        pltpu.VMEM((2, 2 * CH, CPS), jnp.float32),
        pltpu.VMEM((ZR, CPS), jnp.float32),
        pltpu.VMEM_SHARED((NP, CPS), jnp.float32),
        pltpu.SemaphoreType.DMA((2,)),
    ],
    compiler_params=_sc_params,
)
def _agg_kernel(ghf_hbm, srcb_hbm, dst_hbm, agg_hbm,
                sidx_v, didx_v, rows_v, zv, acc_sh, gsem):
    cid = lax.axis_index("c")
    sid = lax.axis_index("s")

    def _zz(i, _):
        for c in range(CPS // 16):
            zv[i, pl.ds(c * 16, 16)] = jnp.zeros((16,), jnp.float32)
        return 0

    lax.fori_loop(0, ZR, _zz, 0)
    for t in range(RPT // ZR):
        pltpu.sync_copy(zv, acc_sh.at[pl.ds(sid * RPT + t * ZR, ZR)])
    plsc.subcore_barrier()

    pltpu.sync_copy(srcb_hbm.at[cid, sid], sidx_v)
    pltpu.sync_copy(dst_hbm.at[sid], didx_v)

    NSC = TCH // 2

    def _sidx(j):
        return sidx_v.at[j]

    def _didx(j):
        return didx_v.at[j]

    for b in range(2):
        pltpu.async_copy(ghf_hbm.at[_sidx(b)], rows_v.at[b], gsem.at[b])

    def _step(j, b, issue_gather):
        pltpu.make_async_copy(
            ghf_hbm.at[_sidx(j)], rows_v.at[b], gsem.at[b]).wait()
        pltpu.sync_copy(rows_v.at[b], acc_sh.at[_didx(j)], add=True)
        if issue_gather:
            @pl.when(j + 2 < NSC)
            def _():
                pltpu.async_copy(
                    ghf_hbm.at[_sidx(j + 2)], rows_v.at[b], gsem.at[b])

    def _pair(p, _):
        for b in range(2):
            _step(2 * p + b, b, True)
        return 0

    lax.fori_loop(0, NSC // 2, _pair, 0)
    _step(NSC - 1, (NSC - 1) % 2, False)
    plsc.subcore_barrier()

    pltpu.sync_copy(acc_sh.at[pl.ds(sid * RPT, RPT)],
                    agg_hbm.at[cid, pl.ds(sid * RPT, RPT)])


BLK = 2000
GRID = N // BLK


def _dinv(degt_ref):
    deg = degt_ref[:, 0:1] + degt_ref[:, 1:2] + 1.0
    return lax.rsqrt(deg)


def _tc1_body(degt_ref, x_ref, w_ref, g_ref):
    h = jnp.dot(x_ref[...], w_ref[0], preferred_element_type=jnp.float32)
    g_ref[0] = h * _dinv(degt_ref)


def _tc2_body(degt_ref, agg_ref, g_ref, b_ref, w_ref, out_ref):
    dinv = _dinv(degt_ref)
    a = jnp.concatenate([agg_ref[0] + g_ref[0], agg_ref[1] + g_ref[1]], axis=1)
    h1 = jnp.maximum(a * dinv + b_ref[...], 0.0)
    h2 = jnp.dot(h1, w_ref[0], preferred_element_type=jnp.float32)
    out_ref[0] = h2 * dinv


def _tc3_body(degt_ref, agg_ref, g_ref, b_ref, out_ref):
    dinv = _dinv(degt_ref)
    a = jnp.concatenate([agg_ref[0] + g_ref[0], agg_ref[1] + g_ref[1]], axis=1)
    out_ref[...] = a * dinv + b_ref[...]


_deg2_spec = pl.BlockSpec((BLK, NC), lambda j, i: (i, 0))
_deg1_spec = pl.BlockSpec((BLK, NC), lambda i: (i, 0))
_x_spec = pl.BlockSpec((BLK, D), lambda j, i: (i, 0))
_whalf_spec = pl.BlockSpec((1, D, CPS), lambda j, i: (j, 0, 0))
_ghalf_out_spec = pl.BlockSpec((1, BLK, CPS), lambda j, i: (j, i, 0))
_pair2_spec = pl.BlockSpec((NC, BLK, CPS), lambda j, i: (0, i, 0))
_pair1_spec = pl.BlockSpec((NC, BLK, CPS), lambda i: (0, i, 0))
_bias2_spec = pl.BlockSpec((1, D), lambda j, i: (0, 0))
_bias1_spec = pl.BlockSpec((1, D), lambda i: (0, 0))
_out_spec = pl.BlockSpec((BLK, D), lambda i: (i, 0))

_tc1 = pl.pallas_call(
    _tc1_body,
    grid=(NC, GRID),
    in_specs=[_deg2_spec, _x_spec, _whalf_spec],
    out_specs=_ghalf_out_spec,
    out_shape=jax.ShapeDtypeStruct((NC, N, CPS), jnp.float32),
)

_tc2 = pl.pallas_call(
    _tc2_body,
    grid=(NC, GRID),
    in_specs=[_deg2_spec, _pair2_spec, _pair2_spec, _bias2_spec, _whalf_spec],
    out_specs=_ghalf_out_spec,
    out_shape=jax.ShapeDtypeStruct((NC, N, CPS), jnp.float32),
)

_tc3 = pl.pallas_call(
    _tc3_body,
    grid=(GRID,),
    in_specs=[_deg1_spec, _pair1_spec, _pair1_spec, _bias1_spec],
    out_specs=_out_spec,
    out_shape=jax.ShapeDtypeStruct((N, D), jnp.float32),
)


def kernel(x, edge_index, W1, b1, W2, b2):
    ei = edge_index.astype(jnp.int32)
    pad = EP - E
    srcp = jnp.concatenate([ei[0], jnp.zeros((pad,), jnp.int32)])
    dstp = jnp.concatenate(
        [ei[1], N + jnp.arange(pad, dtype=jnp.int32) % (NP - N)])
    src3 = srcp.reshape(NS, TCH // 2, 2 * CH)
    srcb = jnp.stack([src3, src3 + N])
    dst3 = dstp.reshape(NS, TCH // 2, 2 * CH)
    dstw = ei[1].reshape(NW, NCH, DCH)
    b1r = b1.reshape(1, D)
    b2r = b2.reshape(1, D)
    W1s = jnp.stack([W1[:, :CPS], W1[:, CPS:]])
    W2s = jnp.stack([W2[:, :CPS], W2[:, CPS:]])

    degp = _deg_kernel(dstw)
    degt = degp.reshape(NC, NP)[:, :N].T

    g1 = _tc1(degt, x, W1s)
    agg1 = _agg_kernel(g1.reshape(NC * N, CPS), srcb, dst3)
    g2 = _tc2(degt, agg1, g1, b1r, W2s)
    agg2 = _agg_kernel(g2.reshape(NC * N, CPS), srcb, dst3)
    out = _tc3(degt, agg2, g2, b2r)
    return out


# speedup vs baseline: 1.5138x; 1.0255x over previous
import functools

import jax
import jax.numpy as jnp
from jax import lax
from jax.experimental import pallas as pl
from jax.experimental.pallas import tpu as pltpu
from jax.experimental.pallas import tpu_sc as plsc

N = 10000
NP = 10240
D = 128
E = 320000
NC = 2
NS = 16
NW = NC * NS
CPS = D // NC
CH = 256
TCH = 79
EPT = TCH * CH
EP = NS * EPT
DCH = 80
EPW = E // NW
NCH = EPW // DCH
RPT = NP // NS
ZR = 128

_mesh = plsc.VectorSubcoreMesh(core_axis_name="c", subcore_axis_name="s")
_sc_params = pltpu.CompilerParams(use_tc_tiling_on_sc=False)


@functools.partial(
    pl.kernel,
    out_type=jax.ShapeDtypeStruct((NC, 1, NP), jnp.float32),
    mesh=_mesh,
    scratch_types=[
        pltpu.VMEM((DCH,), jnp.float32),
        pltpu.VMEM((NCH, DCH), jnp.int32),
        pltpu.VMEM((RPT,), jnp.float32),
        pltpu.VMEM_SHARED((NP,), jnp.float32),
    ],
    compiler_params=_sc_params,
)
def _deg_kernel(dst_hbm, degp_hbm, ones_v, idx_v, zv, deg_sh):
    cid = lax.axis_index("c")
    sid = lax.axis_index("s")
    wid = sid * NC + cid

    for i in range(DCH // 16):
        ones_v[pl.ds(i * 16, 16)] = jnp.ones((16,), jnp.float32)

    def _zz(i, _):
        zv[pl.ds(i * 16, 16)] = jnp.zeros((16,), jnp.float32)
        return 0

    lax.fori_loop(0, RPT // 16, _zz, 0)
    pltpu.sync_copy(zv, deg_sh.at[pl.ds(sid * RPT, RPT)])
    plsc.subcore_barrier()

    pltpu.sync_copy(dst_hbm.at[wid], idx_v)

    def _body(j, _):
        pltpu.sync_copy(ones_v, deg_sh.at[idx_v.at[j]], add=True)
        return 0

    lax.fori_loop(0, NCH, _body, 0)
    plsc.subcore_barrier()

    pltpu.sync_copy(deg_sh.at[pl.ds(sid * RPT, RPT)],
                    degp_hbm.at[cid, 0, pl.ds(sid * RPT, RPT)])


@functools.partial(
    pl.kernel,
    out_type=jax.ShapeDtypeStruct((NC, NP, CPS), jnp.float32),
    mesh=_mesh,
    scratch_types=[
        pltpu.VMEM((TCH, CH), jnp.int32),
        pltpu.VMEM((TCH, CH), jnp.int32),
        pltpu.VMEM((2, CH, CPS), jnp.float32),
        pltpu.VMEM((ZR, CPS), jnp.float32),
        pltpu.VMEM_SHARED((NP, CPS), jnp.float32),
        pltpu.SemaphoreType.DMA((2,)),
    ],
    compiler_params=_sc_params,
)
def _agg_kernel(ghf_hbm, srcb_hbm, dst_hbm, agg_hbm,
                sidx_v, didx_v, rows_v, zv, acc_sh, gsem):
    cid = lax.axis_index("c")
    sid = lax.axis_index("s")

    def _zz(i, _):
        for c in range(CPS // 16):
            zv[i, pl.ds(c * 16, 16)] = jnp.zeros((16,), jnp.float32)
        return 0

    lax.fori_loop(0, ZR, _zz, 0)
    for t in range(RPT // ZR):
        pltpu.sync_copy(zv, acc_sh.at[pl.ds(sid * RPT + t * ZR, ZR)])
    plsc.subcore_barrier()

    pltpu.sync_copy(srcb_hbm.at[cid, sid], sidx_v)
    pltpu.sync_copy(dst_hbm.at[sid], didx_v)

    NSC = TCH

    def _sidx(j):
        return sidx_v.at[j]

    def _didx(j):
        return didx_v.at[j]

    for b in range(2):
        pltpu.async_copy(ghf_hbm.at[_sidx(b)], rows_v.at[b], gsem.at[b])

    def _step(j, b, issue_gather):
        pltpu.make_async_copy(
            ghf_hbm.at[_sidx(j)], rows_v.at[b], gsem.at[b]).wait()
        pltpu.sync_copy(rows_v.at[b], acc_sh.at[_didx(j)], add=True)
        if issue_gather:
            @pl.when(j + 2 < NSC)
            def _():
                pltpu.async_copy(
                    ghf_hbm.at[_sidx(j + 2)], rows_v.at[b], gsem.at[b])

    def _pair(p, _):
        for b in range(2):
            _step(2 * p + b, b, True)
        return 0

    lax.fori_loop(0, NSC // 2, _pair, 0)
    _step(NSC - 1, (NSC - 1) % 2, False)
    plsc.subcore_barrier()

    pltpu.sync_copy(acc_sh.at[pl.ds(sid * RPT, RPT)],
                    agg_hbm.at[cid, pl.ds(sid * RPT, RPT)])


BLK = 2000
GRID = N // BLK


def _dinv(degt_ref):
    deg = degt_ref[:, 0:1] + degt_ref[:, 1:2] + 1.0
    return lax.rsqrt(deg)


def _tc1_body(degt_ref, x_ref, w_ref, g_ref):
    dinv = _dinv(degt_ref)
    h = jnp.dot(x_ref[...], w_ref[...], preferred_element_type=jnp.float32)
    g_ref[0] = h[:, :CPS] * dinv
    g_ref[1] = h[:, CPS:] * dinv


def _tc2_body(degt_ref, agg_ref, g_ref, b_ref, w_ref, out_ref):
    dinv = _dinv(degt_ref)
    a = jnp.concatenate([agg_ref[0] + g_ref[0], agg_ref[1] + g_ref[1]], axis=1)
    h1 = jnp.maximum(a * dinv + b_ref[...], 0.0)
    h2 = jnp.dot(h1, w_ref[...], preferred_element_type=jnp.float32)
    out_ref[0] = h2[:, :CPS] * dinv
    out_ref[1] = h2[:, CPS:] * dinv


def _tc3_body(degt_ref, agg_ref, g_ref, b_ref, out_ref):
    dinv = _dinv(degt_ref)
    a = jnp.concatenate([agg_ref[0] + g_ref[0], agg_ref[1] + g_ref[1]], axis=1)
    out_ref[...] = a * dinv + b_ref[...]


_deg_spec = pl.BlockSpec((BLK, NC), lambda i: (i, 0))
_x_spec = pl.BlockSpec((BLK, D), lambda i: (i, 0))
_w_spec = pl.BlockSpec((D, D), lambda i: (0, 0))
_pair_spec = pl.BlockSpec((NC, BLK, CPS), lambda i: (0, i, 0))
_bias_spec = pl.BlockSpec((1, D), lambda i: (0, 0))
_out_spec = pl.BlockSpec((BLK, D), lambda i: (i, 0))

_tc1 = pl.pallas_call(
    _tc1_body,
    grid=(GRID,),
    in_specs=[_deg_spec, _x_spec, _w_spec],
    out_specs=_pair_spec,
    out_shape=jax.ShapeDtypeStruct((NC, N, CPS), jnp.float32),
)

_tc2 = pl.pallas_call(
    _tc2_body,
    grid=(GRID,),
    in_specs=[_deg_spec, _pair_spec, _pair_spec, _bias_spec, _w_spec],
    out_specs=_pair_spec,
    out_shape=jax.ShapeDtypeStruct((NC, N, CPS), jnp.float32),
)

_tc3 = pl.pallas_call(
    _tc3_body,
    grid=(GRID,),
    in_specs=[_deg_spec, _pair_spec, _pair_spec, _bias_spec],
    out_specs=_out_spec,
    out_shape=jax.ShapeDtypeStruct((N, D), jnp.float32),
)


def kernel(x, edge_index, W1, b1, W2, b2):
    ei = edge_index.astype(jnp.int32)
    pad = EP - E
    srcp = jnp.concatenate([ei[0], jnp.zeros((pad,), jnp.int32)])
    dstp = jnp.concatenate(
        [ei[1], N + jnp.arange(pad, dtype=jnp.int32) % (NP - N)])
    src3 = srcp.reshape(NS, TCH, CH)
    srcb = jnp.stack([src3, src3 + N])
    dst3 = dstp.reshape(NS, TCH, CH)
    dstw = ei[1].reshape(NW, NCH, DCH)
    b1r = b1.reshape(1, D)
    b2r = b2.reshape(1, D)

    degp = _deg_kernel(dstw)
    degt = degp.reshape(NC, NP)[:, :N].T

    g1 = _tc1(degt, x, W1)
    agg1 = _agg_kernel(g1.reshape(NC * N, CPS), srcb, dst3)
    g2 = _tc2(degt, agg1, g1, b1r, W2)
    agg2 = _agg_kernel(g2.reshape(NC * N, CPS), srcb, dst3)
    out = _tc3(degt, agg2, g2, b2r)
    return out
